# Initial kernel scaffold; baseline (speedup 1.0000x reference)
import jax
import jax.numpy as jnp
from jax.experimental import pallas as pl


def kernel(state, x, res_vals, res_rows, res_cols, res_bias, in_vals, in_rows, in_cols):
    raise NotImplementedError("write your pallas kernel here")


# speedup vs baseline: 3.5280x; 3.5280x over previous
import functools

import jax
import jax.numpy as jnp
from jax import lax
from jax.experimental import pallas as pl
from jax.experimental.pallas import tpu as pltpu
from jax.experimental.pallas import tpu_sc as plsc

N_RES = 4096
N_IN = 512
C_TOT = N_RES + N_IN
NW = 32
SLAB = 24
PASSES = C_TOT // (NW * SLAB)
CHUNK = 8192
LANES = 16


def _densify(idx, vals, zslab):
    n_chunks = idx.shape[0] // CHUNK
    mesh = plsc.VectorSubcoreMesh(core_axis_name="c", subcore_axis_name="s")

    @functools.partial(
        pl.kernel,
        mesh=mesh,
        out_type=jax.ShapeDtypeStruct((C_TOT * N_RES,), jnp.float32),
        scratch_types=[
            pltpu.VMEM((SLAB * N_RES,), jnp.float32),
            pltpu.VMEM((CHUNK,), jnp.int32),
            pltpu.VMEM((CHUNK,), jnp.float32),
        ],
        compiler_params=pltpu.CompilerParams(needs_layout_passes=False),
    )
    def k(idx_hbm, val_hbm, z_hbm, w_hbm, acc, idxb, valb):
        wid = lax.axis_index("s") * 2 + lax.axis_index("c")

        def one_pass(p, carry):
            base = (p * NW + wid) * (SLAB * N_RES)
            pltpu.sync_copy(z_hbm, acc)

            def one_chunk(kc, carry):
                pltpu.sync_copy(idx_hbm.at[pl.ds(kc * CHUNK, CHUNK)], idxb)
                pltpu.sync_copy(val_hbm.at[pl.ds(kc * CHUNK, CHUNK)], valb)

                def one_vec(j, carry):
                    iv = idxb[pl.ds(j * LANES, LANES)]
                    vv = valb[pl.ds(j * LANES, LANES)]
                    loc = iv - base
                    m = plsc.bitcast(loc, jnp.uint32) < jnp.uint32(SLAB * N_RES)
                    loc = jnp.where(m, loc, 0)
                    plsc.addupdate_scatter(acc, [loc], vv, mask=m)
                    return carry

                return lax.fori_loop(0, CHUNK // LANES, one_vec, carry)

            carry = lax.fori_loop(0, n_chunks, one_chunk, carry)
            pltpu.sync_copy(acc, w_hbm.at[pl.ds(base, SLAB * N_RES)])
            return carry

        lax.fori_loop(0, PASSES, one_pass, 0)

    return k(idx, vals, zslab)


def _erf(x):
    ax = jnp.abs(x)
    t = 1.0 / (1.0 + 0.3275911 * ax)
    poly = t * (0.254829592 + t * (-0.284496736 + t * (
        1.421413741 + t * (-1.453152027 + t * 1.061405429))))
    y = 1.0 - poly * jnp.exp(-ax * ax)
    return jnp.where(x < 0, -y, y)


BM = 512
BN = 512


def _mm_kernel(a_ref, w_ref, b_ref, o_ref):
    z = lax.dot_general(
        a_ref[...], w_ref[...], (((1,), (0,)), ((), ())),
        precision=lax.Precision.HIGHEST,
        preferred_element_type=jnp.float32)
    o_ref[...] = _erf(z + b_ref[...])


def _matmul_erf(a, w, bias2):
    m = a.shape[0]
    return pl.pallas_call(
        _mm_kernel,
        grid=(m // BM, N_RES // BN),
        in_specs=[
            pl.BlockSpec((BM, C_TOT), lambda i, j: (i, 0)),
            pl.BlockSpec((C_TOT, BN), lambda i, j: (0, j)),
            pl.BlockSpec((1, BN), lambda i, j: (0, j)),
        ],
        out_specs=pl.BlockSpec((BM, BN), lambda i, j: (i, j)),
        out_shape=jax.ShapeDtypeStruct((m, N_RES), jnp.float32),
        compiler_params=pltpu.CompilerParams(
            dimension_semantics=("parallel", "parallel")),
    )(a, w, bias2)


def kernel(state, x, res_vals, res_rows, res_cols, res_bias,
           in_vals, in_rows, in_cols):
    in_rows = in_rows.astype(jnp.int32)
    in_cols = in_cols.astype(jnp.int32)
    res_rows = res_rows.astype(jnp.int32)
    res_cols = res_cols.astype(jnp.int32)
    idx = jnp.concatenate([
        in_cols * N_RES + in_rows,
        (res_cols + N_IN) * N_RES + res_rows,
    ])
    vals = jnp.concatenate([in_vals, res_vals])
    n = idx.shape[0]
    n_pad = ((n + CHUNK - 1) // CHUNK) * CHUNK
    idx = jnp.pad(idx, (0, n_pad - n))
    vals = jnp.pad(vals, (0, n_pad - n))
    zslab = jnp.zeros((SLAB * N_RES,), jnp.float32)

    w = _densify(idx, vals, zslab).reshape(C_TOT, N_RES)

    a = jnp.concatenate([x, state], axis=1)
    bias2 = res_bias.reshape(1, N_RES)
    return _matmul_erf(a, w, bias2)


# speedup vs baseline: 5.4835x; 1.5543x over previous
import functools

import jax
import jax.numpy as jnp
from jax import lax
from jax.experimental import pallas as pl
from jax.experimental.pallas import tpu as pltpu
from jax.experimental.pallas import tpu_sc as plsc

N_RES = 4096
N_IN = 512
C_TOT = N_RES + N_IN
NW = 32
SLAB = 24
PASSES = C_TOT // (NW * SLAB)
CHUNK = 4096
LANES = 16


def _densify(comb, zslab):
    n_chunks = comb.shape[0] // (2 * CHUNK)
    mesh = plsc.VectorSubcoreMesh(core_axis_name="c", subcore_axis_name="s")

    @functools.partial(
        pl.kernel,
        mesh=mesh,
        out_type=jax.ShapeDtypeStruct((C_TOT * N_RES,), jnp.float32),
        scratch_types=[
            pltpu.VMEM((SLAB * N_RES,), jnp.float32),
            pltpu.VMEM((2 * CHUNK,), jnp.int32),
            pltpu.VMEM((2 * CHUNK,), jnp.int32),
            pltpu.SemaphoreType.DMA,
            pltpu.SemaphoreType.DMA,
        ],
        compiler_params=pltpu.CompilerParams(needs_layout_passes=False),
    )
    def k(comb_hbm, z_hbm, w_hbm, acc, buf0, buf1, sem0, sem1):
        wid = lax.axis_index("s") * 2 + lax.axis_index("c")
        bufs = (buf0, buf1)
        sems = (sem0, sem1)

        def start(kc, b):
            pltpu.async_copy(
                comb_hbm.at[pl.ds(kc * (2 * CHUNK), 2 * CHUNK)], bufs[b],
                sems[b])

        def wait(b):
            pltpu.make_async_copy(
                comb_hbm.at[pl.ds(0, 2 * CHUNK)], bufs[b], sems[b]).wait()

        def process(b, base):
            buf = bufs[b]

            def one_vec(j, carry):
                iv = buf[pl.ds(j * LANES, LANES)]
                vv = plsc.bitcast(buf[pl.ds(CHUNK + j * LANES, LANES)],
                                  jnp.float32)
                loc = iv - base
                m = plsc.bitcast(loc, jnp.uint32) < jnp.uint32(SLAB * N_RES)
                loc = jnp.where(m, loc, 0)
                plsc.addupdate_scatter(acc, [loc], vv, mask=m)
                return carry

            lax.fori_loop(0, CHUNK // LANES, one_vec, 0, unroll=8)

        def one_pass(p, carry):
            base = (p * NW + wid) * (SLAB * N_RES)
            pltpu.sync_copy(z_hbm, acc)
            start(0, 0)
            start(1, 1)

            def one_pair(g, carry):
                for b in range(2):
                    wait(b)
                    process(b, base)

                    @pl.when(2 * g + 2 + b < n_chunks)
                    def _():
                        start(2 * g + 2 + b, b)
                return carry

            carry = lax.fori_loop(0, n_chunks // 2, one_pair, carry)
            pltpu.sync_copy(acc, w_hbm.at[pl.ds(base, SLAB * N_RES)])
            return carry

        lax.fori_loop(0, PASSES, one_pass, 0)

    return k(comb, zslab)


def _erf(x):
    ax = jnp.abs(x)
    t = 1.0 / (1.0 + 0.3275911 * ax)
    poly = t * (0.254829592 + t * (-0.284496736 + t * (
        1.421413741 + t * (-1.453152027 + t * 1.061405429))))
    y = 1.0 - poly * jnp.exp(-ax * ax)
    return jnp.where(x < 0, -y, y)


BM = 512
BN = 512


def _mm_kernel(a_ref, w_ref, b_ref, o_ref):
    z = lax.dot_general(
        a_ref[...], w_ref[...], (((1,), (0,)), ((), ())),
        precision=lax.Precision.HIGHEST,
        preferred_element_type=jnp.float32)
    o_ref[...] = _erf(z + b_ref[...])


def _matmul_erf(a, w, bias2):
    m = a.shape[0]
    return pl.pallas_call(
        _mm_kernel,
        grid=(m // BM, N_RES // BN),
        in_specs=[
            pl.BlockSpec((BM, C_TOT), lambda i, j: (i, 0)),
            pl.BlockSpec((C_TOT, BN), lambda i, j: (0, j)),
            pl.BlockSpec((1, BN), lambda i, j: (0, j)),
        ],
        out_specs=pl.BlockSpec((BM, BN), lambda i, j: (i, j)),
        out_shape=jax.ShapeDtypeStruct((m, N_RES), jnp.float32),
        compiler_params=pltpu.CompilerParams(
            dimension_semantics=("parallel", "parallel")),
    )(a, w, bias2)


def kernel(state, x, res_vals, res_rows, res_cols, res_bias,
           in_vals, in_rows, in_cols):
    in_rows = in_rows.astype(jnp.int32)
    in_cols = in_cols.astype(jnp.int32)
    res_rows = res_rows.astype(jnp.int32)
    res_cols = res_cols.astype(jnp.int32)
    idx = jnp.concatenate([
        in_cols * N_RES + in_rows,
        (res_cols + N_IN) * N_RES + res_rows,
    ])
    vals = jnp.concatenate([in_vals, res_vals])
    n = idx.shape[0]
    n_pad = ((n + 2 * CHUNK - 1) // (2 * CHUNK)) * (2 * CHUNK)
    idx = jnp.pad(idx, (0, n_pad - n))
    vals = jnp.pad(vals, (0, n_pad - n))
    comb = jnp.stack([
        idx.reshape(-1, CHUNK),
        lax.bitcast_convert_type(vals, jnp.int32).reshape(-1, CHUNK),
    ], axis=1).reshape(-1)
    zslab = jnp.zeros((SLAB * N_RES,), jnp.float32)

    w = _densify(comb, zslab).reshape(C_TOT, N_RES)

    a = jnp.concatenate([x, state], axis=1)
    bias2 = res_bias.reshape(1, N_RES)
    return _matmul_erf(a, w, bias2)


# speedup vs baseline: 43.7810x; 7.9842x over previous
import functools

import jax
import jax.numpy as jnp
from jax import lax
from jax.experimental import pallas as pl
from jax.experimental.pallas import tpu as pltpu
from jax.experimental.pallas import tpu_sc as plsc

N_RES = 4096
N_IN = 512
C_TOT = N_RES + N_IN
NW = 32
LANES = 16

B3 = 288
CAP = 96
ROW = 2 * CAP
RND = 8192
ROUNDS = 8
N_PAD3 = NW * ROUNDS * RND
PAD_IDX = C_TOT * N_RES
SCR_WORDS = B3 * NW * ROUNDS * ROW
BPT = B3 // NW


def _mesh():
    return plsc.VectorSubcoreMesh(core_axis_name="c", subcore_axis_name="s")


def _partition(comb):
    @functools.partial(
        pl.kernel,
        mesh=_mesh(),
        out_type=jax.ShapeDtypeStruct((SCR_WORDS,), jnp.int32),
        scratch_types=[
            pltpu.VMEM((B3 * ROW,), jnp.int32),
            pltpu.VMEM((304,), jnp.int32),
            pltpu.VMEM((2 * RND,), jnp.int32),
            pltpu.VMEM((2 * RND,), jnp.int32),
            pltpu.SemaphoreType.DMA,
            pltpu.SemaphoreType.DMA,
            pltpu.SemaphoreType.DMA,
        ],
        compiler_params=pltpu.CompilerParams(needs_layout_passes=False),
    )
    def k(comb_hbm, scr_hbm, stage, cnt, buf0, buf1, sem0, sem1, fsem):
        wid = lax.axis_index("s") * 2 + lax.axis_index("c")
        zeros16 = jnp.zeros((LANES,), jnp.int32)
        ones16 = jnp.ones((LANES,), jnp.int32)
        bufs, sems = (buf0, buf1), (sem0, sem1)

        def zero_stage(i, c):
            stage[pl.ds(i * LANES, LANES)] = zeros16
            return c

        lax.fori_loop(0, B3 * ROW // LANES, zero_stage, 0)

        def start(rr, bb):
            chunk = wid * ROUNDS + rr
            pltpu.async_copy(
                comb_hbm.at[pl.ds(chunk * 2 * RND, 2 * RND)], bufs[bb],
                sems[bb])

        def wait_stream(bb):
            pltpu.make_async_copy(
                comb_hbm.at[pl.ds(0, 2 * RND)], bufs[bb], sems[bb]).wait()

        def drain_flush(i, c):
            pltpu.make_async_copy(
                stage.at[pl.ds(0, ROW)], scr_hbm.at[pl.ds(0, ROW)],
                fsem).wait()
            return c

        start(0, 0)
        start(1, 1)
        for r in range(ROUNDS):
            bb = r % 2
            if r > 0:
                lax.fori_loop(0, B3, drain_flush, 0)

                def zero_vals(bk, c):
                    for j in range(CAP // LANES):
                        stage[pl.ds(bk * ROW + CAP + j * LANES, LANES)] = (
                            zeros16)
                    return c

                lax.fori_loop(0, B3, zero_vals, 0)

            def zero_cnt(i, c):
                cnt[pl.ds(i * LANES, LANES)] = zeros16
                return c

            lax.fori_loop(0, 304 // LANES, zero_cnt, 0)
            wait_stream(bb)
            buf = bufs[bb]

            def vec(j, c):
                iv = buf[pl.ds(j * LANES, LANES)]
                vvb = buf[pl.ds(RND + j * LANES, LANES)]
                bkt = lax.shift_right_logical(iv, 16)
                loc = lax.bitwise_and(iv, jnp.int32(0xFFFF))
                rank = plsc.scan_count(bkt)[0]
                cnts = plsc.load_gather(cnt, [bkt])
                slot = cnts + rank - 1
                valid = (bkt < B3) & (slot < CAP)
                addr = jnp.where(valid, bkt * ROW + slot, 0)
                plsc.store_scatter(stage, [addr], loc, mask=valid)
                plsc.store_scatter(stage, [addr + CAP], vvb, mask=valid)
                plsc.addupdate_scatter(cnt, [bkt], ones16, mask=valid)
                return c

            lax.fori_loop(0, RND // LANES, vec, 0, unroll=2)
            if r + 2 < ROUNDS:
                start(r + 2, bb)

            def flush(bk, c):
                dst = ((bk * NW + wid) * ROUNDS + r) * ROW
                pltpu.async_copy(
                    stage.at[pl.ds(bk * ROW, ROW)],
                    scr_hbm.at[pl.ds(dst, ROW)], fsem)
                return c

            lax.fori_loop(0, B3, flush, 0)
        lax.fori_loop(0, B3, drain_flush, 0)

    return k(comb)


def _scatter_dense(scr, zslab):
    seg_half = NW * ROUNDS * ROW // 2

    @functools.partial(
        pl.kernel,
        mesh=_mesh(),
        out_type=jax.ShapeDtypeStruct((C_TOT * N_RES,), jnp.float32),
        scratch_types=[
            pltpu.VMEM((16 * N_RES,), jnp.float32),
            pltpu.VMEM((seg_half,), jnp.int32),
            pltpu.VMEM((seg_half,), jnp.int32),
            pltpu.SemaphoreType.DMA,
            pltpu.SemaphoreType.DMA,
        ],
        compiler_params=pltpu.CompilerParams(needs_layout_passes=False),
    )
    def k(scr_hbm, z_hbm, w_hbm, acc, sb0, sb1, sem0, sem1):
        wid = lax.axis_index("s") * 2 + lax.axis_index("c")
        sbufs, sems = (sb0, sb1), (sem0, sem1)
        for kb in range(BPT):
            b = wid * BPT + kb
            base = b * 2 * seg_half
            for h in range(2):
                pltpu.async_copy(
                    scr_hbm.at[pl.ds(base + h * seg_half, seg_half)],
                    sbufs[h], sems[h])
            pltpu.sync_copy(z_hbm, acc)
            for h in range(2):
                sbuf = sbufs[h]
                pltpu.make_async_copy(
                    scr_hbm.at[pl.ds(0, seg_half)], sbuf, sems[h]).wait()

                def seg(s, c):
                    for j in range(CAP // LANES):
                        locv = sbuf[pl.ds(s * ROW + j * LANES, LANES)]
                        vvb = sbuf[pl.ds(s * ROW + CAP + j * LANES, LANES)]
                        m = vvb != 0
                        plsc.addupdate_scatter(
                            acc, [locv], plsc.bitcast(vvb, jnp.float32),
                            mask=m)
                    return c

                lax.fori_loop(0, seg_half // ROW, seg, 0)
            pltpu.sync_copy(acc, w_hbm.at[pl.ds(b * 16 * N_RES, 16 * N_RES)])

    return k(scr, zslab)


def _erf(x):
    ax = jnp.abs(x)
    t = 1.0 / (1.0 + 0.3275911 * ax)
    poly = t * (0.254829592 + t * (-0.284496736 + t * (
        1.421413741 + t * (-1.453152027 + t * 1.061405429))))
    y = 1.0 - poly * jnp.exp(-ax * ax)
    return jnp.where(x < 0, -y, y)


BM = 512
BN = 512


def _mm_kernel(a_ref, w_ref, b_ref, o_ref):
    z = lax.dot_general(
        a_ref[...], w_ref[...], (((1,), (0,)), ((), ())),
        precision=lax.Precision.HIGHEST,
        preferred_element_type=jnp.float32)
    o_ref[...] = _erf(z + b_ref[...])


def _matmul_erf(a, w, bias2):
    m = a.shape[0]
    return pl.pallas_call(
        _mm_kernel,
        grid=(m // BM, N_RES // BN),
        in_specs=[
            pl.BlockSpec((BM, C_TOT), lambda i, j: (i, 0)),
            pl.BlockSpec((C_TOT, BN), lambda i, j: (0, j)),
            pl.BlockSpec((1, BN), lambda i, j: (0, j)),
        ],
        out_specs=pl.BlockSpec((BM, BN), lambda i, j: (i, j)),
        out_shape=jax.ShapeDtypeStruct((m, N_RES), jnp.float32),
        compiler_params=pltpu.CompilerParams(
            dimension_semantics=("parallel", "parallel")),
    )(a, w, bias2)


def kernel(state, x, res_vals, res_rows, res_cols, res_bias,
           in_vals, in_rows, in_cols):
    in_rows = in_rows.astype(jnp.int32)
    in_cols = in_cols.astype(jnp.int32)
    res_rows = res_rows.astype(jnp.int32)
    res_cols = res_cols.astype(jnp.int32)
    idx = jnp.concatenate([
        in_cols * N_RES + in_rows,
        (res_cols + N_IN) * N_RES + res_rows,
    ])
    vals = jnp.concatenate([in_vals, res_vals])
    n = idx.shape[0]
    idx = jnp.pad(idx, (0, N_PAD3 - n), constant_values=PAD_IDX)
    vals = jnp.pad(vals, (0, N_PAD3 - n))
    comb = jnp.stack([
        idx.reshape(-1, RND),
        lax.bitcast_convert_type(vals, jnp.int32).reshape(-1, RND),
    ], axis=1).reshape(-1)
    zslab = jnp.zeros((16 * N_RES,), jnp.float32)

    scr = _partition(comb)
    w = _scatter_dense(scr, zslab).reshape(C_TOT, N_RES)

    a = jnp.concatenate([x, state], axis=1)
    bias2 = res_bias.reshape(1, N_RES)
    return _matmul_erf(a, w, bias2)
